# baseline (speedup 1.0000x reference)
import functools

import jax
import jax.numpy as jnp
from jax import lax
from jax.experimental import pallas as pl
from jax.experimental.pallas import tpu as pltpu
from jax.experimental.pallas import tpu_sc as plsc

EPS = 1e-8
TAU = 1.5
LC, LS, LA = 0.1, 0.05, 0.01
K = 8
NEG = -1e30
BIG = 2 ** 30
LOG2 = 0.6931471805599453

def _norm_body(z_ref, o_ref):
    z = z_ref[...]
    nrm = jnp.sqrt(jnp.sum(z * z, axis=1, keepdims=True))
    o_ref[...] = z / jnp.maximum(nrm, 1e-12)


def _normalize(z):
    n, d = z.shape
    bm = 1024
    return pl.pallas_call(
        _norm_body,
        grid=(n // bm,),
        in_specs=[pl.BlockSpec((bm, d), lambda i: (i, 0))],
        out_specs=pl.BlockSpec((bm, d), lambda i: (i, 0)),
        out_shape=jax.ShapeDtypeStruct((n, d), jnp.float32),
    )(z)


def _gbar_body(g_ref, gbar_ref, entg_ref, tab_ref):
    g = g_ref[...]
    gb = jnp.clip(jnp.mean(g, axis=0, keepdims=True), EPS, None)
    gbar_ref[...] = gb
    entg_ref[...] = jnp.sum(gb * jnp.log(gb), axis=1, keepdims=True)
    pc = jnp.clip(g, EPS, None)
    ent = jnp.sum(pc * jnp.log(pc), axis=1, keepdims=True)
    e = g.shape[1]
    tab_ref[...] = jnp.concatenate(
        [pc, jnp.broadcast_to(ent, (g.shape[0], e))], axis=1)


def _gbar(gates):
    n, e = gates.shape
    return pl.pallas_call(
        _gbar_body,
        out_shape=(jax.ShapeDtypeStruct((1, e), jnp.float32),
                   jax.ShapeDtypeStruct((1, 1), jnp.float32),
                   jax.ShapeDtypeStruct((n, 2 * e), jnp.float32)),
    )(gates)


CH = 32


def _chunk_top8(w, bn):
    colf = lax.broadcasted_iota(jnp.int32, (1, bn), 1).astype(jnp.float32)
    m = jnp.max(w, axis=1, keepdims=True)
    vals, poss = [], []
    for _ in range(K):
        c = w == m
        pos = jnp.min(jnp.where(c, colf, 2e9), axis=1, keepdims=True)
        vals.append(m)
        poss.append(pos)
        w = jnp.where(c, NEG, w)
        m = jnp.max(w, axis=1, keepdims=True)
    return jnp.concatenate(vals, axis=1), jnp.concatenate(poss, axis=1)


def _merge_top8(av, ai, bv, bi, bm):
    cv = jnp.concatenate([av, bv], axis=1)
    ci = jnp.concatenate([ai, bi], axis=1)
    colf = lax.broadcasted_iota(jnp.int32, (1, 2 * K), 1).astype(jnp.float32)
    vals, idxs = [], []
    for _ in range(K):
        m = jnp.max(cv, axis=1, keepdims=True)
        pos = jnp.min(jnp.where(cv == m, colf, 2e9), axis=1, keepdims=True)
        sel = colf == pos
        vals.append(m)
        idxs.append(jnp.sum(jnp.where(sel, ci, 0.0), axis=1, keepdims=True))
        cv = jnp.where(sel, NEG, cv)
    return jnp.concatenate(vals, axis=1), jnp.concatenate(idxs, axis=1)


def _topk_body(bm, bn, zl_ref, zr_ref, val_ref, idx_ref, rv_ref, ri_ref,
               sim_ref, tv_ref, ti_ref):
    i = pl.program_id(0)
    j = pl.program_id(1)
    nj = pl.num_programs(1)
    sim = lax.dot_general(zl_ref[...], zr_ref[...], (((1,), (1,)), ((), ())),
                          preferred_element_type=jnp.float32)
    on_diag = (i * bm) // bn == j

    @pl.when(on_diag)
    def _():
        rows = i * bm + lax.broadcasted_iota(jnp.int32, (bm, bn), 0)
        cols = j * bn + lax.broadcasted_iota(jnp.int32, (bm, bn), 1)
        sim_ref[...] = jnp.where(rows == cols, 0.0, sim)

    @pl.when(jnp.logical_not(on_diag))
    def _():
        sim_ref[...] = sim

    def chunk(c, _):
        w = sim_ref[pl.ds(c * CH, CH), :]
        tv, tp = _chunk_top8(w, bn)
        tv_ref[pl.ds(c * CH, CH), :] = tv
        ti_ref[pl.ds(c * CH, CH), :] = tp + jnp.float32(j * bn)
        return _

    lax.fori_loop(0, bm // CH, chunk, None)

    @pl.when(j == 0)
    def _():
        rv_ref[...] = tv_ref[...]
        ri_ref[...] = ti_ref[...]

    @pl.when(j > 0)
    def _():
        nv, ni = _merge_top8(rv_ref[...], ri_ref[...], tv_ref[...],
                             ti_ref[...], bm)
        rv_ref[...] = nv
        ri_ref[...] = ni

    @pl.when(j == nj - 1)
    def _():
        val_ref[...] = rv_ref[...]
        idx_ref[...] = ri_ref[...].astype(jnp.int32)


def _knn_top8(z_norm):
    n, d = z_norm.shape
    bm, bn = 256, 1024
    return pl.pallas_call(
        functools.partial(_topk_body, bm, bn),
        grid=(n // bm, n // bn),
        in_specs=[pl.BlockSpec((bm, d), lambda i, j: (i, 0)),
                  pl.BlockSpec((bn, d), lambda i, j: (j, 0))],
        out_specs=(pl.BlockSpec((bm, K), lambda i, j: (i, 0)),
                   pl.BlockSpec((bm, K), lambda i, j: (i, 0))),
        out_shape=(jax.ShapeDtypeStruct((n, K), jnp.float32),
                   jax.ShapeDtypeStruct((n, K), jnp.int32)),
        scratch_shapes=[pltpu.VMEM((bm, K), jnp.float32),
                        pltpu.VMEM((bm, K), jnp.float32),
                        pltpu.VMEM((bm, bn), jnp.float32),
                        pltpu.VMEM((bm, K), jnp.float32),
                        pltpu.VMEM((bm, K), jnp.float32)],
        compiler_params=pltpu.CompilerParams(
            dimension_semantics=("arbitrary", "arbitrary")),
    )(z_norm, z_norm)


def _gather_rows(table, idx):
    b = idx.shape[0]
    v, e = table.shape
    nw = 32
    chunk = 128
    b_per_w = b // nw
    nch = b_per_w // chunk
    mesh = plsc.VectorSubcoreMesh(core_axis_name="c", subcore_axis_name="s")

    @functools.partial(
        pl.kernel, mesh=mesh,
        out_type=jax.ShapeDtypeStruct((b, e), jnp.float32),
        scratch_types=[pltpu.VMEM((chunk,), jnp.int32),
                       pltpu.VMEM((chunk, e), jnp.float32),
                       pltpu.SemaphoreType.DMA],
    )
    def gather_k(table_hbm, idx_hbm, out_hbm, idx_v, rows_v, sem):
        wid = lax.axis_index("s") * 2 + lax.axis_index("c")
        base = wid * b_per_w
        for c in range(nch):
            off = base + c * chunk
            pltpu.sync_copy(idx_hbm.at[pl.ds(off, chunk)], idx_v)
            pltpu.async_copy(table_hbm.at[idx_v], rows_v, sem).wait()
            pltpu.sync_copy(rows_v, out_hbm.at[pl.ds(off, chunk)])

    return gather_k(table, idx)


def _loss_body(bm, n, e, t_ref, gj_ref, kv_ref, ti_ref, gbar_ref, entg_ref,
               out_ref, coh_s, ali_s, cnt_s):
    i = pl.program_id(0)
    ni = pl.num_programs(0)

    @pl.when(i == 0)
    def _():
        coh_s[...] = jnp.zeros((1, 1), jnp.float32)
        ali_s[...] = jnp.zeros((1, 1), jnp.float32)
        cnt_s[...] = jnp.zeros((1, e), jnp.float32)

    tab = t_ref[...]
    p = tab[:, :e]
    entp = tab[:, e:e + 1]
    gjt = gj_ref[...]
    q = gjt[:, :e]
    entq = gjt[:, e:e + 1].reshape(bm, K)
    s = p[:, None, :] + q.reshape(bm, K, e)
    slogs = jnp.sum(s * jnp.log(s), axis=2)
    ssum = jnp.sum(s, axis=2)
    js = 0.5 * (entp + entq - slogs + LOG2 * ssum)
    coh_s[...] += jnp.sum(kv_ref[...] * js, keepdims=True).reshape(1, 1)

    gb = gbar_ref[...]
    s2 = p + gb
    slogs2 = jnp.sum(s2 * jnp.log(s2), axis=1, keepdims=True)
    ssum2 = jnp.sum(s2, axis=1, keepdims=True)
    js2 = 0.5 * (entp + entg_ref[...] - slogs2 + LOG2 * ssum2)
    ali_s[...] += jnp.sum(js2, keepdims=True).reshape(1, 1)

    t0 = ti_ref[...][:, 0:1]
    eids = lax.broadcasted_iota(jnp.int32, (1, e), 1)
    cnt_s[...] += jnp.sum(jnp.where(t0 == eids, 1.0, 0.0), axis=0,
                          keepdims=True)

    @pl.when(i == ni - 1)
    def _():
        l_coh = coh_s[...] / (n * K)
        l_ali = ali_s[...] / n
        cnts = cnt_s[...]
        n_bar = jnp.maximum(jnp.sum(cnts, keepdims=True) / e, EPS)
        over = cnts / n_bar - TAU
        l_sep = jnp.sum(jnp.maximum(over, 0.0) ** 2, keepdims=True) / e
        loss = LC * l_coh + LS * l_sep + LA * l_ali
        out_ref[...] = jnp.concatenate([l_coh, l_sep, l_ali, loss], axis=1)


def _losses(table, gj, knn_val, topk_idx, gbar, entg):
    n = table.shape[0]
    e = table.shape[1] // 2
    bm = 512
    return pl.pallas_call(
        functools.partial(_loss_body, bm, n, e),
        grid=(n // bm,),
        in_specs=[pl.BlockSpec((bm, 2 * e), lambda i: (i, 0)),
                  pl.BlockSpec((bm * K, 2 * e), lambda i: (i, 0)),
                  pl.BlockSpec((bm, K), lambda i: (i, 0)),
                  pl.BlockSpec((bm, K), lambda i: (i, 0)),
                  pl.BlockSpec((1, e), lambda i: (0, 0)),
                  pl.BlockSpec((1, 1), lambda i: (0, 0))],
        out_specs=pl.BlockSpec((1, 4), lambda i: (0, 0)),
        out_shape=jax.ShapeDtypeStruct((1, 4), jnp.float32),
        scratch_shapes=[pltpu.VMEM((1, 1), jnp.float32),
                        pltpu.VMEM((1, 1), jnp.float32),
                        pltpu.VMEM((1, e), jnp.float32)],
        compiler_params=pltpu.CompilerParams(
            dimension_semantics=("arbitrary",)),
    )(table, gj, knn_val, topk_idx, gbar, entg)


def kernel(z, gates_soft, topk_idx, num_experts):
    del num_experts
    z_norm = _normalize(z)
    gbar, entg, table = _gbar(gates_soft)
    knn_val, knn_idx = _knn_top8(z_norm)
    gj = _gather_rows(table, knn_idx.reshape(-1))
    out = _losses(table, gj, knn_val, topk_idx, gbar, entg)
    return out.reshape(4)


# speedup vs baseline: 2.0512x; 2.0512x over previous
import functools

import jax
import jax.numpy as jnp
from jax import lax
from jax.experimental import pallas as pl
from jax.experimental.pallas import tpu as pltpu
from jax.experimental.pallas import tpu_sc as plsc

EPS = 1e-8
TAU = 1.5
LC, LS, LA = 0.1, 0.05, 0.01
K = 8
NEG = -1e30
BIG = 2 ** 30
LOG2 = 0.6931471805599453

def _norm_body(z_ref, o_ref):
    z = z_ref[...]
    nrm = jnp.sqrt(jnp.sum(z * z, axis=1, keepdims=True))
    o_ref[...] = z / jnp.maximum(nrm, 1e-12)


def _normalize(z):
    n, d = z.shape
    bm = 1024
    return pl.pallas_call(
        _norm_body,
        grid=(n // bm,),
        in_specs=[pl.BlockSpec((bm, d), lambda i: (i, 0))],
        out_specs=pl.BlockSpec((bm, d), lambda i: (i, 0)),
        out_shape=jax.ShapeDtypeStruct((n, d), jnp.float32),
    )(z)


def _gbar_body(g_ref, gbar_ref, entg_ref, tab_ref):
    g = g_ref[...]
    gb = jnp.clip(jnp.mean(g, axis=0, keepdims=True), EPS, None)
    gbar_ref[...] = gb
    entg_ref[...] = jnp.sum(gb * jnp.log(gb), axis=1, keepdims=True)
    pc = jnp.clip(g, EPS, None)
    ent = jnp.sum(pc * jnp.log(pc), axis=1, keepdims=True)
    e = g.shape[1]
    tab_ref[...] = jnp.concatenate(
        [pc, jnp.broadcast_to(ent, (g.shape[0], e))], axis=1)


def _gbar(gates):
    n, e = gates.shape
    return pl.pallas_call(
        _gbar_body,
        out_shape=(jax.ShapeDtypeStruct((1, e), jnp.float32),
                   jax.ShapeDtypeStruct((1, 1), jnp.float32),
                   jax.ShapeDtypeStruct((n, 2 * e), jnp.float32)),
    )(gates)


def _merge_top8(av, ai, bv, bi, bm):
    cv = jnp.concatenate([av, bv], axis=1)
    ci = jnp.concatenate([ai, bi], axis=1)
    colf = lax.broadcasted_iota(jnp.int32, (1, 2 * K), 1).astype(jnp.float32)
    vals, idxs = [], []
    for _ in range(K):
        m = jnp.max(cv, axis=1, keepdims=True)
        pos = jnp.min(jnp.where(cv == m, colf, 2e9), axis=1, keepdims=True)
        sel = colf == pos
        vals.append(m)
        idxs.append(jnp.sum(jnp.where(sel, ci, 0.0), axis=1, keepdims=True))
        cv = jnp.where(sel, NEG, cv)
    return jnp.concatenate(vals, axis=1), jnp.concatenate(idxs, axis=1)


def _topk_body(bm, bn, zl_ref, zr_ref, val_ref, idx_ref, rv_ref, ri_ref,
               sim_ref):
    i = pl.program_id(0)
    j = pl.program_id(1)
    nj = pl.num_programs(1)
    sim = lax.dot_general(zl_ref[...], zr_ref[...], (((1,), (1,)), ((), ())),
                          preferred_element_type=jnp.float32)
    on_diag = (i * bm) // bn == j

    @pl.when(on_diag)
    def _():
        rows = i * bm + lax.broadcasted_iota(jnp.int32, (bm, bn), 0)
        cols = j * bn + lax.broadcasted_iota(jnp.int32, (bm, bn), 1)
        sim_ref[...] = jnp.where(rows == cols, 0.0, sim)

    @pl.when(jnp.logical_not(on_diag))
    def _():
        sim_ref[...] = sim

    w = sim_ref[...]
    colf = lax.broadcasted_iota(jnp.int32, (1, bn), 1).astype(jnp.float32)
    m = jnp.max(w, axis=1, keepdims=True)
    vals, poss = [], []
    for _ in range(K):
        pos = jnp.min(jnp.where(w == m, colf, 2e9), axis=1, keepdims=True)
        vals.append(m)
        poss.append(pos)
        w = jnp.where(w == m, NEG, w)
        m = jnp.max(w, axis=1, keepdims=True)
    tv = jnp.concatenate(vals, axis=1)
    ti = jnp.concatenate(poss, axis=1) + jnp.float32(j * bn)

    @pl.when(j == 0)
    def _():
        rv_ref[...] = tv
        ri_ref[...] = ti

    @pl.when(j > 0)
    def _():
        nv, ni = _merge_top8(rv_ref[...], ri_ref[...], tv, ti, bm)
        rv_ref[...] = nv
        ri_ref[...] = ni

    @pl.when(j == nj - 1)
    def _():
        val_ref[...] = rv_ref[...]
        idx_ref[...] = ri_ref[...].astype(jnp.int32)


def _knn_top8(z_norm):
    n, d = z_norm.shape
    bm, bn = 256, 1024
    return pl.pallas_call(
        functools.partial(_topk_body, bm, bn),
        grid=(n // bm, n // bn),
        in_specs=[pl.BlockSpec((bm, d), lambda i, j: (i, 0)),
                  pl.BlockSpec((bn, d), lambda i, j: (j, 0))],
        out_specs=(pl.BlockSpec((bm, K), lambda i, j: (i, 0)),
                   pl.BlockSpec((bm, K), lambda i, j: (i, 0))),
        out_shape=(jax.ShapeDtypeStruct((n, K), jnp.float32),
                   jax.ShapeDtypeStruct((n, K), jnp.int32)),
        scratch_shapes=[pltpu.VMEM((bm, K), jnp.float32),
                        pltpu.VMEM((bm, K), jnp.float32),
                        pltpu.VMEM((bm, bn), jnp.float32)],
        compiler_params=pltpu.CompilerParams(
            dimension_semantics=("arbitrary", "arbitrary")),
    )(z_norm, z_norm)


def _gather_rows(table, idx):
    b = idx.shape[0]
    v, e = table.shape
    nw = 32
    chunk = 128
    b_per_w = b // nw
    nch = b_per_w // chunk
    mesh = plsc.VectorSubcoreMesh(core_axis_name="c", subcore_axis_name="s")

    @functools.partial(
        pl.kernel, mesh=mesh,
        out_type=jax.ShapeDtypeStruct((b, e), jnp.float32),
        scratch_types=[pltpu.VMEM((chunk,), jnp.int32),
                       pltpu.VMEM((chunk, e), jnp.float32),
                       pltpu.SemaphoreType.DMA],
    )
    def gather_k(table_hbm, idx_hbm, out_hbm, idx_v, rows_v, sem):
        wid = lax.axis_index("s") * 2 + lax.axis_index("c")
        base = wid * b_per_w
        for c in range(nch):
            off = base + c * chunk
            pltpu.sync_copy(idx_hbm.at[pl.ds(off, chunk)], idx_v)
            pltpu.async_copy(table_hbm.at[idx_v], rows_v, sem).wait()
            pltpu.sync_copy(rows_v, out_hbm.at[pl.ds(off, chunk)])

    return gather_k(table, idx)


def _loss_body(bm, n, e, t_ref, gj_ref, kv_ref, ti_ref, gbar_ref, entg_ref,
               out_ref, coh_s, ali_s, cnt_s):
    i = pl.program_id(0)
    ni = pl.num_programs(0)

    @pl.when(i == 0)
    def _():
        coh_s[...] = jnp.zeros((1, 1), jnp.float32)
        ali_s[...] = jnp.zeros((1, 1), jnp.float32)
        cnt_s[...] = jnp.zeros((1, e), jnp.float32)

    tab = t_ref[...]
    p = tab[:, :e]
    entp = tab[:, e:e + 1]
    gjt = gj_ref[...]
    q = gjt[:, :e]
    entq = gjt[:, e:e + 1].reshape(bm, K)
    s = p[:, None, :] + q.reshape(bm, K, e)
    slogs = jnp.sum(s * jnp.log(s), axis=2)
    ssum = jnp.sum(s, axis=2)
    js = 0.5 * (entp + entq - slogs + LOG2 * ssum)
    coh_s[...] += jnp.sum(kv_ref[...] * js, keepdims=True).reshape(1, 1)

    gb = gbar_ref[...]
    s2 = p + gb
    slogs2 = jnp.sum(s2 * jnp.log(s2), axis=1, keepdims=True)
    ssum2 = jnp.sum(s2, axis=1, keepdims=True)
    js2 = 0.5 * (entp + entg_ref[...] - slogs2 + LOG2 * ssum2)
    ali_s[...] += jnp.sum(js2, keepdims=True).reshape(1, 1)

    t0 = ti_ref[...][:, 0:1]
    eids = lax.broadcasted_iota(jnp.int32, (1, e), 1)
    cnt_s[...] += jnp.sum(jnp.where(t0 == eids, 1.0, 0.0), axis=0,
                          keepdims=True)

    @pl.when(i == ni - 1)
    def _():
        l_coh = coh_s[...] / (n * K)
        l_ali = ali_s[...] / n
        cnts = cnt_s[...]
        n_bar = jnp.maximum(jnp.sum(cnts, keepdims=True) / e, EPS)
        over = cnts / n_bar - TAU
        l_sep = jnp.sum(jnp.maximum(over, 0.0) ** 2, keepdims=True) / e
        loss = LC * l_coh + LS * l_sep + LA * l_ali
        out_ref[...] = jnp.concatenate([l_coh, l_sep, l_ali, loss], axis=1)


def _losses(table, gj, knn_val, topk_idx, gbar, entg):
    n = table.shape[0]
    e = table.shape[1] // 2
    bm = 512
    return pl.pallas_call(
        functools.partial(_loss_body, bm, n, e),
        grid=(n // bm,),
        in_specs=[pl.BlockSpec((bm, 2 * e), lambda i: (i, 0)),
                  pl.BlockSpec((bm * K, 2 * e), lambda i: (i, 0)),
                  pl.BlockSpec((bm, K), lambda i: (i, 0)),
                  pl.BlockSpec((bm, K), lambda i: (i, 0)),
                  pl.BlockSpec((1, e), lambda i: (0, 0)),
                  pl.BlockSpec((1, 1), lambda i: (0, 0))],
        out_specs=pl.BlockSpec((1, 4), lambda i: (0, 0)),
        out_shape=jax.ShapeDtypeStruct((1, 4), jnp.float32),
        scratch_shapes=[pltpu.VMEM((1, 1), jnp.float32),
                        pltpu.VMEM((1, 1), jnp.float32),
                        pltpu.VMEM((1, e), jnp.float32)],
        compiler_params=pltpu.CompilerParams(
            dimension_semantics=("arbitrary",)),
    )(table, gj, knn_val, topk_idx, gbar, entg)


def kernel(z, gates_soft, topk_idx, num_experts):
    del num_experts
    z_norm = _normalize(z)
    gbar, entg, table = _gbar(gates_soft)
    knn_val, knn_idx = _knn_top8(z_norm)
    gj = _gather_rows(table, knn_idx.reshape(-1))
    out = _losses(table, gj, knn_val, topk_idx, gbar, entg)
    return out.reshape(4)


# speedup vs baseline: 2.7427x; 1.3371x over previous
import functools

import jax
import jax.numpy as jnp
from jax import lax
from jax.experimental import pallas as pl
from jax.experimental.pallas import tpu as pltpu
from jax.experimental.pallas import tpu_sc as plsc

EPS = 1e-8
TAU = 1.5
LC, LS, LA = 0.1, 0.05, 0.01
K = 8
NEG = -1e30
BIG = 2 ** 30
LOG2 = 0.6931471805599453

def _norm_body(z_ref, o_ref):
    z = z_ref[...]
    nrm = jnp.sqrt(jnp.sum(z * z, axis=1, keepdims=True))
    o_ref[...] = z / jnp.maximum(nrm, 1e-12)


def _normalize(z):
    n, d = z.shape
    bm = 1024
    return pl.pallas_call(
        _norm_body,
        grid=(n // bm,),
        in_specs=[pl.BlockSpec((bm, d), lambda i: (i, 0))],
        out_specs=pl.BlockSpec((bm, d), lambda i: (i, 0)),
        out_shape=jax.ShapeDtypeStruct((n, d), jnp.float32),
    )(z)


def _gbar_body(g_ref, gbar_ref, entg_ref, tab_ref):
    g = g_ref[...]
    gb = jnp.clip(jnp.mean(g, axis=0, keepdims=True), EPS, None)
    gbar_ref[...] = gb
    entg_ref[...] = jnp.sum(gb * jnp.log(gb), axis=1, keepdims=True)
    pc = jnp.clip(g, EPS, None)
    ent = jnp.sum(pc * jnp.log(pc), axis=1, keepdims=True)
    e = g.shape[1]
    tab_ref[...] = jnp.concatenate(
        [pc, jnp.broadcast_to(ent, (g.shape[0], e))], axis=1)


def _gbar(gates):
    n, e = gates.shape
    return pl.pallas_call(
        _gbar_body,
        out_shape=(jax.ShapeDtypeStruct((1, e), jnp.float32),
                   jax.ShapeDtypeStruct((1, 1), jnp.float32),
                   jax.ShapeDtypeStruct((n, 2 * e), jnp.float32)),
    )(gates)


def _merge_top8(av, ai, bv, bi, bm):
    cv = jnp.concatenate([av, bv], axis=1)
    ci = jnp.concatenate([ai, bi], axis=1)
    colf = lax.broadcasted_iota(jnp.int32, (1, 2 * K), 1).astype(jnp.float32)
    vals, idxs = [], []
    for _ in range(K):
        m = jnp.max(cv, axis=1, keepdims=True)
        pos = jnp.min(jnp.where(cv == m, colf, 2e9), axis=1, keepdims=True)
        sel = colf == pos
        vals.append(m)
        idxs.append(jnp.sum(jnp.where(sel, ci, 0.0), axis=1, keepdims=True))
        cv = jnp.where(sel, NEG, cv)
    return jnp.concatenate(vals, axis=1), jnp.concatenate(idxs, axis=1)


def _topk_body(bm, bn, zl_ref, zr_ref, val_ref, idx_ref, rv_ref, ri_ref,
               sim_ref):
    i = pl.program_id(0)
    j = pl.program_id(1)
    nj = pl.num_programs(1)
    sim = lax.dot_general(zl_ref[...], zr_ref[...], (((1,), (1,)), ((), ())),
                          preferred_element_type=jnp.float32)
    on_diag = (i * bm) // bn == j

    @pl.when(on_diag)
    def _():
        rows = i * bm + lax.broadcasted_iota(jnp.int32, (bm, bn), 0)
        cols = j * bn + lax.broadcasted_iota(jnp.int32, (bm, bn), 1)
        sim_ref[...] = jnp.where(rows == cols, 0.0, sim)

    @pl.when(jnp.logical_not(on_diag))
    def _():
        sim_ref[...] = sim

    w = sim_ref[...]
    colf = lax.broadcasted_iota(jnp.int32, (1, bn), 1).astype(jnp.float32)
    m = jnp.max(w, axis=1, keepdims=True)
    vals, poss = [], []
    for _ in range(K):
        pos = jnp.min(jnp.where(w == m, colf, 2e9), axis=1, keepdims=True)
        vals.append(m)
        poss.append(pos)
        w = jnp.where(w == m, NEG, w)
        m = jnp.max(w, axis=1, keepdims=True)
    tv = jnp.concatenate(vals, axis=1)
    ti = jnp.concatenate(poss, axis=1) + jnp.float32(j * bn)

    @pl.when(j == 0)
    def _():
        rv_ref[...] = tv
        ri_ref[...] = ti

    @pl.when(j > 0)
    def _():
        nv, ni = _merge_top8(rv_ref[...], ri_ref[...], tv, ti, bm)
        rv_ref[...] = nv
        ri_ref[...] = ni

    @pl.when(j == nj - 1)
    def _():
        val_ref[...] = rv_ref[...]
        idx_ref[...] = ri_ref[...].astype(jnp.int32)


def _knn_top8(z_norm):
    n, d = z_norm.shape
    bm, bn = 256, 2048
    return pl.pallas_call(
        functools.partial(_topk_body, bm, bn),
        grid=(n // bm, n // bn),
        in_specs=[pl.BlockSpec((bm, d), lambda i, j: (i, 0)),
                  pl.BlockSpec((bn, d), lambda i, j: (j, 0))],
        out_specs=(pl.BlockSpec((bm, K), lambda i, j: (i, 0)),
                   pl.BlockSpec((bm, K), lambda i, j: (i, 0))),
        out_shape=(jax.ShapeDtypeStruct((n, K), jnp.float32),
                   jax.ShapeDtypeStruct((n, K), jnp.int32)),
        scratch_shapes=[pltpu.VMEM((bm, K), jnp.float32),
                        pltpu.VMEM((bm, K), jnp.float32),
                        pltpu.VMEM((bm, bn), jnp.float32)],
        compiler_params=pltpu.CompilerParams(
            dimension_semantics=("arbitrary", "arbitrary")),
    )(z_norm, z_norm)


def _gather_rows(table, idx):
    b = idx.shape[0]
    v, e = table.shape
    nw = 32
    chunk = 128
    b_per_w = b // nw
    nch = b_per_w // chunk
    mesh = plsc.VectorSubcoreMesh(core_axis_name="c", subcore_axis_name="s")

    @functools.partial(
        pl.kernel, mesh=mesh,
        out_type=jax.ShapeDtypeStruct((b, e), jnp.float32),
        scratch_types=[pltpu.VMEM((chunk,), jnp.int32),
                       pltpu.VMEM((chunk, e), jnp.float32),
                       pltpu.SemaphoreType.DMA],
    )
    def gather_k(table_hbm, idx_hbm, out_hbm, idx_v, rows_v, sem):
        wid = lax.axis_index("s") * 2 + lax.axis_index("c")
        base = wid * b_per_w
        for c in range(nch):
            off = base + c * chunk
            pltpu.sync_copy(idx_hbm.at[pl.ds(off, chunk)], idx_v)
            pltpu.async_copy(table_hbm.at[idx_v], rows_v, sem).wait()
            pltpu.sync_copy(rows_v, out_hbm.at[pl.ds(off, chunk)])

    return gather_k(table, idx)


def _loss_body(bm, n, e, t_ref, gj_ref, kv_ref, ti_ref, gbar_ref, entg_ref,
               out_ref, coh_s, ali_s, cnt_s):
    i = pl.program_id(0)
    ni = pl.num_programs(0)

    @pl.when(i == 0)
    def _():
        coh_s[...] = jnp.zeros((1, 1), jnp.float32)
        ali_s[...] = jnp.zeros((1, 1), jnp.float32)
        cnt_s[...] = jnp.zeros((1, e), jnp.float32)

    tab = t_ref[...]
    p = tab[:, :e]
    entp = tab[:, e:e + 1]
    gjt = gj_ref[...]
    q = gjt[:, :e]
    entq = gjt[:, e:e + 1].reshape(bm, K)
    s = p[:, None, :] + q.reshape(bm, K, e)
    slogs = jnp.sum(s * jnp.log(s), axis=2)
    ssum = jnp.sum(s, axis=2)
    js = 0.5 * (entp + entq - slogs + LOG2 * ssum)
    coh_s[...] += jnp.sum(kv_ref[...] * js, keepdims=True).reshape(1, 1)

    gb = gbar_ref[...]
    s2 = p + gb
    slogs2 = jnp.sum(s2 * jnp.log(s2), axis=1, keepdims=True)
    ssum2 = jnp.sum(s2, axis=1, keepdims=True)
    js2 = 0.5 * (entp + entg_ref[...] - slogs2 + LOG2 * ssum2)
    ali_s[...] += jnp.sum(js2, keepdims=True).reshape(1, 1)

    t0 = ti_ref[...][:, 0:1]
    eids = lax.broadcasted_iota(jnp.int32, (1, e), 1)
    cnt_s[...] += jnp.sum(jnp.where(t0 == eids, 1.0, 0.0), axis=0,
                          keepdims=True)

    @pl.when(i == ni - 1)
    def _():
        l_coh = coh_s[...] / (n * K)
        l_ali = ali_s[...] / n
        cnts = cnt_s[...]
        n_bar = jnp.maximum(jnp.sum(cnts, keepdims=True) / e, EPS)
        over = cnts / n_bar - TAU
        l_sep = jnp.sum(jnp.maximum(over, 0.0) ** 2, keepdims=True) / e
        loss = LC * l_coh + LS * l_sep + LA * l_ali
        out_ref[...] = jnp.concatenate([l_coh, l_sep, l_ali, loss], axis=1)


def _losses(table, gj, knn_val, topk_idx, gbar, entg):
    n = table.shape[0]
    e = table.shape[1] // 2
    bm = 512
    return pl.pallas_call(
        functools.partial(_loss_body, bm, n, e),
        grid=(n // bm,),
        in_specs=[pl.BlockSpec((bm, 2 * e), lambda i: (i, 0)),
                  pl.BlockSpec((bm * K, 2 * e), lambda i: (i, 0)),
                  pl.BlockSpec((bm, K), lambda i: (i, 0)),
                  pl.BlockSpec((bm, K), lambda i: (i, 0)),
                  pl.BlockSpec((1, e), lambda i: (0, 0)),
                  pl.BlockSpec((1, 1), lambda i: (0, 0))],
        out_specs=pl.BlockSpec((1, 4), lambda i: (0, 0)),
        out_shape=jax.ShapeDtypeStruct((1, 4), jnp.float32),
        scratch_shapes=[pltpu.VMEM((1, 1), jnp.float32),
                        pltpu.VMEM((1, 1), jnp.float32),
                        pltpu.VMEM((1, e), jnp.float32)],
        compiler_params=pltpu.CompilerParams(
            dimension_semantics=("arbitrary",)),
    )(table, gj, knn_val, topk_idx, gbar, entg)


def kernel(z, gates_soft, topk_idx, num_experts):
    del num_experts
    z_norm = _normalize(z)
    gbar, entg, table = _gbar(gates_soft)
    knn_val, knn_idx = _knn_top8(z_norm)
    gj = _gather_rows(table, knn_idx.reshape(-1))
    out = _losses(table, gj, knn_val, topk_idx, gbar, entg)
    return out.reshape(4)


# speedup vs baseline: 3.2284x; 1.1771x over previous
import functools

import jax
import jax.numpy as jnp
from jax import lax
from jax.experimental import pallas as pl
from jax.experimental.pallas import tpu as pltpu
from jax.experimental.pallas import tpu_sc as plsc

EPS = 1e-8
TAU = 1.5
LC, LS, LA = 0.1, 0.05, 0.01
K = 8
NEG = -1e30
BIG = 2 ** 30
LOG2 = 0.6931471805599453

def _norm_body(z_ref, o_ref):
    z = z_ref[...]
    nrm = jnp.sqrt(jnp.sum(z * z, axis=1, keepdims=True))
    o_ref[...] = z / jnp.maximum(nrm, 1e-12)


def _normalize(z):
    n, d = z.shape
    bm = 1024
    return pl.pallas_call(
        _norm_body,
        grid=(n // bm,),
        in_specs=[pl.BlockSpec((bm, d), lambda i: (i, 0))],
        out_specs=pl.BlockSpec((bm, d), lambda i: (i, 0)),
        out_shape=jax.ShapeDtypeStruct((n, d), jnp.float32),
    )(z)


def _gbar_body(g_ref, gbar_ref, entg_ref, tab_ref):
    g = g_ref[...]
    gb = jnp.clip(jnp.mean(g, axis=0, keepdims=True), EPS, None)
    gbar_ref[...] = gb
    entg_ref[...] = jnp.sum(gb * jnp.log(gb), axis=1, keepdims=True)
    pc = jnp.clip(g, EPS, None)
    ent = jnp.sum(pc * jnp.log(pc), axis=1, keepdims=True)
    e = g.shape[1]
    tab_ref[...] = jnp.concatenate(
        [pc, jnp.broadcast_to(ent, (g.shape[0], e))], axis=1)


def _gbar(gates):
    n, e = gates.shape
    return pl.pallas_call(
        _gbar_body,
        out_shape=(jax.ShapeDtypeStruct((1, e), jnp.float32),
                   jax.ShapeDtypeStruct((1, 1), jnp.float32),
                   jax.ShapeDtypeStruct((n, 2 * e), jnp.float32)),
    )(gates)


def _merge_top8(av, ai, bv, bi, bm):
    cv = jnp.concatenate([av, bv], axis=1)
    ci = jnp.concatenate([ai, bi], axis=1)
    colf = lax.broadcasted_iota(jnp.int32, (1, 2 * K), 1).astype(jnp.float32)
    vals, idxs = [], []
    for _ in range(K):
        m = jnp.max(cv, axis=1, keepdims=True)
        pos = jnp.min(jnp.where(cv == m, colf, 2e9), axis=1, keepdims=True)
        sel = colf == pos
        vals.append(m)
        idxs.append(jnp.sum(jnp.where(sel, ci, 0.0), axis=1, keepdims=True))
        cv = jnp.where(sel, NEG, cv)
    return jnp.concatenate(vals, axis=1), jnp.concatenate(idxs, axis=1)


def _topk_body(bm, bn, zl_ref, zr_ref, val_ref, idx_ref, rv_ref, ri_ref,
               sim_ref):
    i = pl.program_id(0)
    j = pl.program_id(1)
    nj = pl.num_programs(1)
    sim = lax.dot_general(zl_ref[...], zr_ref[...], (((1,), (1,)), ((), ())),
                          preferred_element_type=jnp.float32)
    on_diag = (i * bm) // bn == j

    @pl.when(on_diag)
    def _():
        rows = i * bm + lax.broadcasted_iota(jnp.int32, (bm, bn), 0)
        cols = j * bn + lax.broadcasted_iota(jnp.int32, (bm, bn), 1)
        sim_ref[...] = jnp.where(rows == cols, 0.0, sim)

    @pl.when(jnp.logical_not(on_diag))
    def _():
        sim_ref[...] = sim

    w = sim_ref[...]
    colf = lax.broadcasted_iota(jnp.int32, (1, bn), 1).astype(jnp.float32)
    m = jnp.max(w, axis=1, keepdims=True)
    vals, poss = [], []
    for _ in range(K):
        pos = jnp.min(jnp.where(w == m, colf, 2e9), axis=1, keepdims=True)
        vals.append(m)
        poss.append(pos)
        w = jnp.where(w == m, NEG, w)
        m = jnp.max(w, axis=1, keepdims=True)
    tv = jnp.concatenate(vals, axis=1)
    ti = jnp.concatenate(poss, axis=1) + jnp.float32(j * bn)

    @pl.when(j == 0)
    def _():
        rv_ref[...] = tv
        ri_ref[...] = ti

    @pl.when(j > 0)
    def _():
        nv, ni = _merge_top8(rv_ref[...], ri_ref[...], tv, ti, bm)
        rv_ref[...] = nv
        ri_ref[...] = ni

    @pl.when(j == nj - 1)
    def _():
        val_ref[...] = rv_ref[...]
        idx_ref[...] = ri_ref[...].astype(jnp.int32)


def _knn_top8(z_norm):
    n, d = z_norm.shape
    bm, bn = 256, 4096
    return pl.pallas_call(
        functools.partial(_topk_body, bm, bn),
        grid=(n // bm, n // bn),
        in_specs=[pl.BlockSpec((bm, d), lambda i, j: (i, 0)),
                  pl.BlockSpec((bn, d), lambda i, j: (j, 0))],
        out_specs=(pl.BlockSpec((bm, K), lambda i, j: (i, 0)),
                   pl.BlockSpec((bm, K), lambda i, j: (i, 0))),
        out_shape=(jax.ShapeDtypeStruct((n, K), jnp.float32),
                   jax.ShapeDtypeStruct((n, K), jnp.int32)),
        scratch_shapes=[pltpu.VMEM((bm, K), jnp.float32),
                        pltpu.VMEM((bm, K), jnp.float32),
                        pltpu.VMEM((bm, bn), jnp.float32)],
        compiler_params=pltpu.CompilerParams(
            dimension_semantics=("arbitrary", "arbitrary")),
    )(z_norm, z_norm)


def _gather_rows(table, idx):
    b = idx.shape[0]
    v, e = table.shape
    nw = 32
    chunk = 128
    b_per_w = b // nw
    nch = b_per_w // chunk
    mesh = plsc.VectorSubcoreMesh(core_axis_name="c", subcore_axis_name="s")

    @functools.partial(
        pl.kernel, mesh=mesh,
        out_type=jax.ShapeDtypeStruct((b, e), jnp.float32),
        scratch_types=[pltpu.VMEM((chunk,), jnp.int32),
                       pltpu.VMEM((chunk, e), jnp.float32),
                       pltpu.SemaphoreType.DMA],
    )
    def gather_k(table_hbm, idx_hbm, out_hbm, idx_v, rows_v, sem):
        wid = lax.axis_index("s") * 2 + lax.axis_index("c")
        base = wid * b_per_w
        for c in range(nch):
            off = base + c * chunk
            pltpu.sync_copy(idx_hbm.at[pl.ds(off, chunk)], idx_v)
            pltpu.async_copy(table_hbm.at[idx_v], rows_v, sem).wait()
            pltpu.sync_copy(rows_v, out_hbm.at[pl.ds(off, chunk)])

    return gather_k(table, idx)


def _loss_body(bm, n, e, t_ref, gj_ref, kv_ref, ti_ref, gbar_ref, entg_ref,
               out_ref, coh_s, ali_s, cnt_s):
    i = pl.program_id(0)
    ni = pl.num_programs(0)

    @pl.when(i == 0)
    def _():
        coh_s[...] = jnp.zeros((1, 1), jnp.float32)
        ali_s[...] = jnp.zeros((1, 1), jnp.float32)
        cnt_s[...] = jnp.zeros((1, e), jnp.float32)

    tab = t_ref[...]
    p = tab[:, :e]
    entp = tab[:, e:e + 1]
    gjt = gj_ref[...]
    q = gjt[:, :e]
    entq = gjt[:, e:e + 1].reshape(bm, K)
    s = p[:, None, :] + q.reshape(bm, K, e)
    slogs = jnp.sum(s * jnp.log(s), axis=2)
    ssum = jnp.sum(s, axis=2)
    js = 0.5 * (entp + entq - slogs + LOG2 * ssum)
    coh_s[...] += jnp.sum(kv_ref[...] * js, keepdims=True).reshape(1, 1)

    gb = gbar_ref[...]
    s2 = p + gb
    slogs2 = jnp.sum(s2 * jnp.log(s2), axis=1, keepdims=True)
    ssum2 = jnp.sum(s2, axis=1, keepdims=True)
    js2 = 0.5 * (entp + entg_ref[...] - slogs2 + LOG2 * ssum2)
    ali_s[...] += jnp.sum(js2, keepdims=True).reshape(1, 1)

    t0 = ti_ref[...][:, 0:1]
    eids = lax.broadcasted_iota(jnp.int32, (1, e), 1)
    cnt_s[...] += jnp.sum(jnp.where(t0 == eids, 1.0, 0.0), axis=0,
                          keepdims=True)

    @pl.when(i == ni - 1)
    def _():
        l_coh = coh_s[...] / (n * K)
        l_ali = ali_s[...] / n
        cnts = cnt_s[...]
        n_bar = jnp.maximum(jnp.sum(cnts, keepdims=True) / e, EPS)
        over = cnts / n_bar - TAU
        l_sep = jnp.sum(jnp.maximum(over, 0.0) ** 2, keepdims=True) / e
        loss = LC * l_coh + LS * l_sep + LA * l_ali
        out_ref[...] = jnp.concatenate([l_coh, l_sep, l_ali, loss], axis=1)


def _losses(table, gj, knn_val, topk_idx, gbar, entg):
    n = table.shape[0]
    e = table.shape[1] // 2
    bm = 512
    return pl.pallas_call(
        functools.partial(_loss_body, bm, n, e),
        grid=(n // bm,),
        in_specs=[pl.BlockSpec((bm, 2 * e), lambda i: (i, 0)),
                  pl.BlockSpec((bm * K, 2 * e), lambda i: (i, 0)),
                  pl.BlockSpec((bm, K), lambda i: (i, 0)),
                  pl.BlockSpec((bm, K), lambda i: (i, 0)),
                  pl.BlockSpec((1, e), lambda i: (0, 0)),
                  pl.BlockSpec((1, 1), lambda i: (0, 0))],
        out_specs=pl.BlockSpec((1, 4), lambda i: (0, 0)),
        out_shape=jax.ShapeDtypeStruct((1, 4), jnp.float32),
        scratch_shapes=[pltpu.VMEM((1, 1), jnp.float32),
                        pltpu.VMEM((1, 1), jnp.float32),
                        pltpu.VMEM((1, e), jnp.float32)],
        compiler_params=pltpu.CompilerParams(
            dimension_semantics=("arbitrary",)),
    )(table, gj, knn_val, topk_idx, gbar, entg)


def kernel(z, gates_soft, topk_idx, num_experts):
    del num_experts
    z_norm = _normalize(z)
    gbar, entg, table = _gbar(gates_soft)
    knn_val, knn_idx = _knn_top8(z_norm)
    gj = _gather_rows(table, knn_idx.reshape(-1))
    out = _losses(table, gj, knn_val, topk_idx, gbar, entg)
    return out.reshape(4)


# speedup vs baseline: 3.7479x; 1.1609x over previous
import functools

import jax
import jax.numpy as jnp
from jax import lax
from jax.experimental import pallas as pl
from jax.experimental.pallas import tpu as pltpu
from jax.experimental.pallas import tpu_sc as plsc

EPS = 1e-8
TAU = 1.5
LC, LS, LA = 0.1, 0.05, 0.01
K = 8
NEG = -1e30
BIG = 2 ** 30
LOG2 = 0.6931471805599453

def _norm_body(z_ref, o_ref):
    z = z_ref[...]
    nrm = jnp.sqrt(jnp.sum(z * z, axis=1, keepdims=True))
    o_ref[...] = z / jnp.maximum(nrm, 1e-12)


def _normalize(z):
    n, d = z.shape
    bm = 1024
    return pl.pallas_call(
        _norm_body,
        grid=(n // bm,),
        in_specs=[pl.BlockSpec((bm, d), lambda i: (i, 0))],
        out_specs=pl.BlockSpec((bm, d), lambda i: (i, 0)),
        out_shape=jax.ShapeDtypeStruct((n, d), jnp.float32),
    )(z)


def _gbar_body(g_ref, gbar_ref, entg_ref, tab_ref):
    g = g_ref[...]
    gb = jnp.clip(jnp.mean(g, axis=0, keepdims=True), EPS, None)
    gbar_ref[...] = gb
    entg_ref[...] = jnp.sum(gb * jnp.log(gb), axis=1, keepdims=True)
    pc = jnp.clip(g, EPS, None)
    ent = jnp.sum(pc * jnp.log(pc), axis=1, keepdims=True)
    e = g.shape[1]
    tab_ref[...] = jnp.concatenate(
        [pc, jnp.broadcast_to(ent, (g.shape[0], e))], axis=1)


def _gbar(gates):
    n, e = gates.shape
    return pl.pallas_call(
        _gbar_body,
        out_shape=(jax.ShapeDtypeStruct((1, e), jnp.float32),
                   jax.ShapeDtypeStruct((1, 1), jnp.float32),
                   jax.ShapeDtypeStruct((n, 2 * e), jnp.float32)),
    )(gates)


def _merge_top8(av, ai, bv, bi, bm):
    cv = jnp.concatenate([av, bv], axis=1)
    ci = jnp.concatenate([ai, bi], axis=1)
    colf = lax.broadcasted_iota(jnp.int32, (1, 2 * K), 1).astype(jnp.float32)
    vals, idxs = [], []
    for _ in range(K):
        m = jnp.max(cv, axis=1, keepdims=True)
        pos = jnp.min(jnp.where(cv == m, colf, 2e9), axis=1, keepdims=True)
        sel = colf == pos
        vals.append(m)
        idxs.append(jnp.sum(jnp.where(sel, ci, 0.0), axis=1, keepdims=True))
        cv = jnp.where(sel, NEG, cv)
    return jnp.concatenate(vals, axis=1), jnp.concatenate(idxs, axis=1)


def _topk_body(bm, bn, zl_ref, zr_ref, val_ref, idx_ref, rv_ref, ri_ref,
               sim_ref):
    i = pl.program_id(0)
    j = pl.program_id(1)
    nj = pl.num_programs(1)
    sim = lax.dot_general(zl_ref[...], zr_ref[...], (((1,), (1,)), ((), ())),
                          preferred_element_type=jnp.float32)
    on_diag = (i * bm) // bn == j

    @pl.when(on_diag)
    def _():
        rows = i * bm + lax.broadcasted_iota(jnp.int32, (bm, bn), 0)
        cols = j * bn + lax.broadcasted_iota(jnp.int32, (bm, bn), 1)
        sim_ref[...] = jnp.where(rows == cols, 0.0, sim)

    @pl.when(jnp.logical_not(on_diag))
    def _():
        sim_ref[...] = sim

    w = sim_ref[...]
    colf = lax.broadcasted_iota(jnp.int32, (1, bn), 1).astype(jnp.float32)
    m = jnp.max(w, axis=1, keepdims=True)
    vals, poss = [], []
    for _ in range(K):
        pos = jnp.min(jnp.where(w == m, colf, 2e9), axis=1, keepdims=True)
        vals.append(m)
        poss.append(pos)
        w = jnp.where(w == m, NEG, w)
        m = jnp.max(w, axis=1, keepdims=True)
    tv = jnp.concatenate(vals, axis=1)
    ti = jnp.concatenate(poss, axis=1) + jnp.float32(j * bn)

    @pl.when(j == 0)
    def _():
        rv_ref[...] = tv
        ri_ref[...] = ti

    @pl.when(j > 0)
    def _():
        nv, ni = _merge_top8(rv_ref[...], ri_ref[...], tv, ti, bm)
        rv_ref[...] = nv
        ri_ref[...] = ni

    @pl.when(j == nj - 1)
    def _():
        val_ref[...] = rv_ref[...]
        idx_ref[...] = ri_ref[...].astype(jnp.int32)


def _knn_top8(z_norm):
    n, d = z_norm.shape
    bm, bn = 256, 8192
    return pl.pallas_call(
        functools.partial(_topk_body, bm, bn),
        grid=(n // bm, n // bn),
        in_specs=[pl.BlockSpec((bm, d), lambda i, j: (i, 0)),
                  pl.BlockSpec((bn, d), lambda i, j: (j, 0))],
        out_specs=(pl.BlockSpec((bm, K), lambda i, j: (i, 0)),
                   pl.BlockSpec((bm, K), lambda i, j: (i, 0))),
        out_shape=(jax.ShapeDtypeStruct((n, K), jnp.float32),
                   jax.ShapeDtypeStruct((n, K), jnp.int32)),
        scratch_shapes=[pltpu.VMEM((bm, K), jnp.float32),
                        pltpu.VMEM((bm, K), jnp.float32),
                        pltpu.VMEM((bm, bn), jnp.float32)],
        compiler_params=pltpu.CompilerParams(
            dimension_semantics=("arbitrary", "arbitrary")),
    )(z_norm, z_norm)


def _gather_rows(table, idx):
    b = idx.shape[0]
    v, e = table.shape
    nw = 32
    chunk = 128
    b_per_w = b // nw
    nch = b_per_w // chunk
    mesh = plsc.VectorSubcoreMesh(core_axis_name="c", subcore_axis_name="s")

    @functools.partial(
        pl.kernel, mesh=mesh,
        out_type=jax.ShapeDtypeStruct((b, e), jnp.float32),
        scratch_types=[pltpu.VMEM((chunk,), jnp.int32),
                       pltpu.VMEM((chunk, e), jnp.float32),
                       pltpu.SemaphoreType.DMA],
    )
    def gather_k(table_hbm, idx_hbm, out_hbm, idx_v, rows_v, sem):
        wid = lax.axis_index("s") * 2 + lax.axis_index("c")
        base = wid * b_per_w
        for c in range(nch):
            off = base + c * chunk
            pltpu.sync_copy(idx_hbm.at[pl.ds(off, chunk)], idx_v)
            pltpu.async_copy(table_hbm.at[idx_v], rows_v, sem).wait()
            pltpu.sync_copy(rows_v, out_hbm.at[pl.ds(off, chunk)])

    return gather_k(table, idx)


def _loss_body(bm, n, e, t_ref, gj_ref, kv_ref, ti_ref, gbar_ref, entg_ref,
               out_ref, coh_s, ali_s, cnt_s):
    i = pl.program_id(0)
    ni = pl.num_programs(0)

    @pl.when(i == 0)
    def _():
        coh_s[...] = jnp.zeros((1, 1), jnp.float32)
        ali_s[...] = jnp.zeros((1, 1), jnp.float32)
        cnt_s[...] = jnp.zeros((1, e), jnp.float32)

    tab = t_ref[...]
    p = tab[:, :e]
    entp = tab[:, e:e + 1]
    gjt = gj_ref[...]
    q = gjt[:, :e]
    entq = gjt[:, e:e + 1].reshape(bm, K)
    s = p[:, None, :] + q.reshape(bm, K, e)
    slogs = jnp.sum(s * jnp.log(s), axis=2)
    ssum = jnp.sum(s, axis=2)
    js = 0.5 * (entp + entq - slogs + LOG2 * ssum)
    coh_s[...] += jnp.sum(kv_ref[...] * js, keepdims=True).reshape(1, 1)

    gb = gbar_ref[...]
    s2 = p + gb
    slogs2 = jnp.sum(s2 * jnp.log(s2), axis=1, keepdims=True)
    ssum2 = jnp.sum(s2, axis=1, keepdims=True)
    js2 = 0.5 * (entp + entg_ref[...] - slogs2 + LOG2 * ssum2)
    ali_s[...] += jnp.sum(js2, keepdims=True).reshape(1, 1)

    t0 = ti_ref[...][:, 0:1]
    eids = lax.broadcasted_iota(jnp.int32, (1, e), 1)
    cnt_s[...] += jnp.sum(jnp.where(t0 == eids, 1.0, 0.0), axis=0,
                          keepdims=True)

    @pl.when(i == ni - 1)
    def _():
        l_coh = coh_s[...] / (n * K)
        l_ali = ali_s[...] / n
        cnts = cnt_s[...]
        n_bar = jnp.maximum(jnp.sum(cnts, keepdims=True) / e, EPS)
        over = cnts / n_bar - TAU
        l_sep = jnp.sum(jnp.maximum(over, 0.0) ** 2, keepdims=True) / e
        loss = LC * l_coh + LS * l_sep + LA * l_ali
        out_ref[...] = jnp.concatenate([l_coh, l_sep, l_ali, loss], axis=1)


def _losses(table, gj, knn_val, topk_idx, gbar, entg):
    n = table.shape[0]
    e = table.shape[1] // 2
    bm = 512
    return pl.pallas_call(
        functools.partial(_loss_body, bm, n, e),
        grid=(n // bm,),
        in_specs=[pl.BlockSpec((bm, 2 * e), lambda i: (i, 0)),
                  pl.BlockSpec((bm * K, 2 * e), lambda i: (i, 0)),
                  pl.BlockSpec((bm, K), lambda i: (i, 0)),
                  pl.BlockSpec((bm, K), lambda i: (i, 0)),
                  pl.BlockSpec((1, e), lambda i: (0, 0)),
                  pl.BlockSpec((1, 1), lambda i: (0, 0))],
        out_specs=pl.BlockSpec((1, 4), lambda i: (0, 0)),
        out_shape=jax.ShapeDtypeStruct((1, 4), jnp.float32),
        scratch_shapes=[pltpu.VMEM((1, 1), jnp.float32),
                        pltpu.VMEM((1, 1), jnp.float32),
                        pltpu.VMEM((1, e), jnp.float32)],
        compiler_params=pltpu.CompilerParams(
            dimension_semantics=("arbitrary",)),
    )(table, gj, knn_val, topk_idx, gbar, entg)


def kernel(z, gates_soft, topk_idx, num_experts):
    del num_experts
    z_norm = _normalize(z)
    gbar, entg, table = _gbar(gates_soft)
    knn_val, knn_idx = _knn_top8(z_norm)
    gj = _gather_rows(table, knn_idx.reshape(-1))
    out = _losses(table, gj, knn_val, topk_idx, gbar, entg)
    return out.reshape(4)
